# baseline (speedup 1.0000x reference)
import functools

import jax
import jax.numpy as jnp
from jax import lax
from jax.experimental import pallas as pl
from jax.experimental.pallas import tpu as pltpu
from jax.experimental.pallas import tpu_sc as plsc

N_E = 8192
D = 256
N_TOK = 16 * 1024
T = 512
G = N_TOK // T

EPS = 1e-12


def _wn_body(w_ref, wn_ref):
    w = w_ref[...]
    n = jnp.sqrt(jnp.sum(w * w, axis=1, keepdims=True))
    wn_ref[...] = w / jnp.maximum(n, EPS)


_wn_call = pl.pallas_call(
    _wn_body,
    out_shape=jax.ShapeDtypeStruct((N_E, D), jnp.float32),
)


def _tc_body(x_ref, wn_ref, dist_ref, idx_ref, usage_ref, cnt_ref):
    i = pl.program_id(0)

    @pl.when(i == 0)
    def _init():
        cnt_ref[...] = jnp.zeros_like(cnt_ref)

    x = x_ref[...]
    xn = x / jnp.maximum(jnp.sqrt(jnp.sum(x * x, axis=1, keepdims=True)), EPS)
    dist = lax.dot_general(xn, wn_ref[...], (((1,), (1,)), ((), ())),
                           preferred_element_type=jnp.float32)
    dist_ref[...] = dist
    rowmax = jnp.max(dist, axis=1, keepdims=True)
    iota_e = lax.broadcasted_iota(jnp.int32, (T, N_E), 1)
    cand = jnp.where(dist == rowmax, iota_e, N_E)
    idx = jnp.min(cand, axis=1).astype(jnp.int32)
    idx_ref[0, 0, :] = idx
    hits = (idx[:, None] == iota_e).astype(jnp.int32)
    cnt_ref[...] += jnp.sum(hits, axis=0, keepdims=True)

    @pl.when(i == G - 1)
    def _fin():
        zero = jnp.sum((cnt_ref[...] == 0).astype(jnp.float32), keepdims=True)
        usage_ref[...] = 100.0 * (zero / N_E)


_tc_call = pl.pallas_call(
    _tc_body,
    grid=(G,),
    in_specs=[
        pl.BlockSpec((T, D), lambda i: (i, 0)),
        pl.BlockSpec((N_E, D), lambda i: (0, 0)),
    ],
    out_specs=[
        pl.BlockSpec((T, N_E), lambda i: (i, 0)),
        pl.BlockSpec((1, 1, T), lambda i: (i, 0, 0)),
        pl.BlockSpec((1, 1), lambda i: (0, 0)),
    ],
    out_shape=[
        jax.ShapeDtypeStruct((N_TOK, N_E), jnp.float32),
        jax.ShapeDtypeStruct((G, 1, T), jnp.int32),
        jax.ShapeDtypeStruct((1, 1), jnp.float32),
    ],
    scratch_shapes=[pltpu.VMEM((1, N_E), jnp.int32)],
)


_SC_INFO = plsc.get_sparse_core_info()
_NC = _SC_INFO.num_cores
_NS = _SC_INFO.num_subcores
_NW = _NC * _NS
_BPW = N_TOK // _NW
_CH = 128
_NCHUNK = _BPW // _CH


@functools.partial(
    pl.kernel,
    mesh=plsc.VectorSubcoreMesh(core_axis_name="c", subcore_axis_name="s"),
    out_type=jax.ShapeDtypeStruct((N_TOK, D), jnp.float32),
    scratch_types=[
        pltpu.VMEM((_CH,), jnp.int32),
        pltpu.VMEM((_CH, D), jnp.float32),
        pltpu.SemaphoreType.DMA,
    ],
)
def _sc_gather(wn_hbm, idx_hbm, out_hbm, idx_v, rows_v, sem):
    wid = lax.axis_index("s") * _NC + lax.axis_index("c")
    base = wid * _BPW
    for c in range(_NCHUNK):
        off = base + c * _CH
        pltpu.sync_copy(idx_hbm.at[pl.ds(off, _CH)], idx_v)
        pltpu.async_copy(wn_hbm.at[idx_v], rows_v, sem).wait()
        pltpu.sync_copy(rows_v, out_hbm.at[pl.ds(off, _CH)])


def kernel(x, weight):
    x = x.astype(jnp.float32)
    xf = x.reshape(N_TOK, D)
    wn = _wn_call(weight)
    dist, idx3, usage = _tc_call(xf, wn)
    idx = idx3.reshape(N_TOK)
    quant = _sc_gather(wn, idx)
    return (
        quant.reshape(16, 1024, D),
        dist.reshape(16, 1024, N_E),
        idx.reshape(16, 1024),
        usage.reshape(()),
    )


# speedup vs baseline: 1.2210x; 1.2210x over previous
import functools

import jax
import jax.numpy as jnp
from jax import lax
from jax.experimental import pallas as pl
from jax.experimental.pallas import tpu as pltpu
from jax.experimental.pallas import tpu_sc as plsc

N_E = 8192
D = 256
N_TOK = 16 * 1024
T = 512
G = N_TOK // T

EPS = 1e-12


def _tc_body(x_ref, w_ref, dist_ref, idx_ref, wn_ref, usage_ref, cnt_ref):
    i = pl.program_id(0)

    @pl.when(i == 0)
    def _init():
        w = w_ref[...]
        n = jnp.sqrt(jnp.sum(w * w, axis=1, keepdims=True))
        wn_ref[...] = w / jnp.maximum(n, EPS)
        cnt_ref[...] = jnp.zeros_like(cnt_ref)

    x = x_ref[...]
    xn = x / jnp.maximum(jnp.sqrt(jnp.sum(x * x, axis=1, keepdims=True)), EPS)
    wn = wn_ref[...]
    dist = lax.dot_general(xn, wn, (((1,), (1,)), ((), ())),
                           preferred_element_type=jnp.float32)
    dist_ref[...] = dist
    idx = jnp.argmax(dist, axis=1).astype(jnp.int32)
    idx_ref[0, 0, :] = idx
    iota_e = lax.broadcasted_iota(jnp.int32, (T, N_E), 1)
    hits = (idx[:, None] == iota_e).astype(jnp.int32)
    cnt_ref[...] += jnp.sum(hits, axis=0, keepdims=True)

    @pl.when(i == G - 1)
    def _fin():
        zero = jnp.sum((cnt_ref[...] == 0).astype(jnp.float32), keepdims=True)
        usage_ref[...] = 100.0 * (zero / N_E)


_tc_call = pl.pallas_call(
    _tc_body,
    grid=(G,),
    in_specs=[
        pl.BlockSpec((T, D), lambda i: (i, 0)),
        pl.BlockSpec((N_E, D), lambda i: (0, 0)),
    ],
    out_specs=[
        pl.BlockSpec((T, N_E), lambda i: (i, 0)),
        pl.BlockSpec((1, 1, T), lambda i: (i, 0, 0)),
        pl.BlockSpec((N_E, D), lambda i: (0, 0)),
        pl.BlockSpec((1, 1), lambda i: (0, 0)),
    ],
    out_shape=[
        jax.ShapeDtypeStruct((N_TOK, N_E), jnp.float32),
        jax.ShapeDtypeStruct((G, 1, T), jnp.int32),
        jax.ShapeDtypeStruct((N_E, D), jnp.float32),
        jax.ShapeDtypeStruct((1, 1), jnp.float32),
    ],
    scratch_shapes=[pltpu.VMEM((1, N_E), jnp.int32)],
)


_SC_INFO = plsc.get_sparse_core_info()
_NC = _SC_INFO.num_cores
_NS = _SC_INFO.num_subcores
_NW = _NC * _NS
_BPW = N_TOK // _NW
_CH = 128
_NCHUNK = _BPW // _CH


@functools.partial(
    pl.kernel,
    mesh=plsc.VectorSubcoreMesh(core_axis_name="c", subcore_axis_name="s"),
    out_type=jax.ShapeDtypeStruct((N_TOK, D), jnp.float32),
    scratch_types=[
        pltpu.VMEM((_CH,), jnp.int32),
        pltpu.VMEM((_CH, D), jnp.float32),
        pltpu.SemaphoreType.DMA,
    ],
)
def _sc_gather(wn_hbm, idx_hbm, out_hbm, idx_v, rows_v, sem):
    wid = lax.axis_index("s") * _NC + lax.axis_index("c")
    base = wid * _BPW
    for c in range(_NCHUNK):
        off = base + c * _CH
        pltpu.sync_copy(idx_hbm.at[pl.ds(off, _CH)], idx_v)
        pltpu.async_copy(wn_hbm.at[idx_v], rows_v, sem).wait()
        pltpu.sync_copy(rows_v, out_hbm.at[pl.ds(off, _CH)])


def kernel(x, weight):
    x = x.astype(jnp.float32)
    xf = x.reshape(N_TOK, D)
    dist, idx3, wn, usage = _tc_call(xf, weight)
    idx = idx3.reshape(N_TOK)
    quant = _sc_gather(wn, idx)
    return (
        quant.reshape(16, 1024, D),
        dist.reshape(16, 1024, N_E),
        idx.reshape(16, 1024),
        usage.reshape(()),
    )
